# baseline (speedup 1.0000x reference)
import functools

import jax
import jax.numpy as jnp
from jax import lax
from jax.experimental import pallas as pl
from jax.experimental.pallas import tpu as pltpu
from jax.experimental.pallas import tpu_sc as plsc

_TH = 0.1
_INV_T = 10.0
_PEAKY_W = 0.5
_REPROJ_W = 1.0
_SCOREMAP_W = 0.5

_B = 2
_N = 256
_H = 128
_W = 128
_D = 200
_M = 400

_NUM_TILES = 32
_HN = _N // 2
_PTS_PER_TILE = (_B * 2 * _N) // _NUM_TILES
_PK_PAD = 1024


def _lane_iota():
    return lax.iota(jnp.int32, 16)


def _sc_body(sim01_r, sim10_r, smap0_r, smap1_r, kp_r,
             s0_r, s1_r, ids0_r, ids1_r, dist_r,
             part_r,
             kpa_v, kpb_v, sa_v, sb_v, wx_v, wy_v,
             sidx_v, midx_v, sva_v, svb_v, mva_v, mvb_v,
             ids0_v, ids1_v, dist_v,
             out_v, sem_a, sem_b, sem_c, sem_d, sem_e, sem_f, sem_g):
    wid = lax.axis_index("c") * 16 + lax.axis_index("s")
    b = wid // 16
    p0 = (wid % 16) * 16
    is_reproj = (wid % 16) < 2
    b_r = b
    h_r = wid % 16

    zf16 = jnp.zeros((16,), jnp.float32)
    for r in range(8):
        out_v[pl.ds(r * 16, 16)] = zf16

    @pl.when(is_reproj)
    def _():
        pltpu.async_copy(dist_r.at[pl.ds(b_r * _N + h_r * _HN, _HN)], dist_v,
                         sem_e)
        pltpu.async_copy(ids0_r, ids0_v, sem_f)
        pltpu.async_copy(ids1_r, ids1_v, sem_g)

    kcp = (pltpu.async_copy(kp_r.at[b], kpa_v, sem_a),
           pltpu.async_copy(kp_r.at[2 + b], kpb_v, sem_b))
    scp = (pltpu.async_copy(s0_r.at[b], sa_v, sem_c),
           pltpu.async_copy(s1_r.at[b], sb_v, sem_d))
    nvec = p0 + _lane_iota()
    nv2 = nvec * 2
    gcp = []
    for c, kp_v in ((0, kpa_v), (1, kpb_v)):
        kcp[c].wait()
        scp[c].wait()
        kx = plsc.load_gather(kp_v, [nv2])
        ky = plsc.load_gather(kp_v, [nv2 + 1])
        x = (kx + 1.0) * (0.5 * (_W - 1))
        y = (ky + 1.0) * (0.5 * (_H - 1))
        x0 = x.astype(jnp.int32)
        y0 = y.astype(jnp.int32)
        wx_v[pl.ds(c * 16, 16)] = x - x0.astype(jnp.float32)
        wy_v[pl.ds(c * 16, 16)] = y - y0.astype(jnp.float32)
        x0c = jnp.clip(x0, 0, _W - 1)
        x1c = jnp.clip(x0 + 1, 0, _W - 1)
        y0c = jnp.clip(y0, 0, _H - 1)
        y1c = jnp.clip(y0 + 1, 0, _H - 1)
        base = (b * _N + nvec) * (_H * _W)
        row0 = base + y0c * _W
        row1 = base + y1c * _W
        sidx_v[pl.ds(c * 64 + 0, 16)] = row0 + x0c
        sidx_v[pl.ds(c * 64 + 16, 16)] = row0 + x1c
        sidx_v[pl.ds(c * 64 + 32, 16)] = row1 + x0c
        sidx_v[pl.ds(c * 64 + 48, 16)] = row1 + x1c
        mrow0 = b * (_H * _W) + y0c * _W
        mrow1 = b * (_H * _W) + y1c * _W
        midx_v[pl.ds(c * 64 + 0, 16)] = mrow0 + x0c
        midx_v[pl.ds(c * 64 + 16, 16)] = mrow0 + x1c
        midx_v[pl.ds(c * 64 + 32, 16)] = mrow1 + x0c
        midx_v[pl.ds(c * 64 + 48, 16)] = mrow1 + x1c
        if c == 0:
            gcp.append(pltpu.async_copy(sim01_r.at[sidx_v.at[pl.ds(0, 64)]],
                                        sva_v, sem_a))
            gcp.append(pltpu.async_copy(smap1_r.at[midx_v.at[pl.ds(0, 64)]],
                                        mva_v, sem_c))
        else:
            gcp.append(pltpu.async_copy(sim10_r.at[sidx_v.at[pl.ds(64, 64)]],
                                        svb_v, sem_b))
            gcp.append(pltpu.async_copy(smap0_r.at[midx_v.at[pl.ds(64, 64)]],
                                        mvb_v, sem_d))
    for c, sv_v, mv_v, sc_v in ((0, sva_v, mva_v, sa_v),
                                (1, svb_v, mvb_v, sb_v)):
        gcp[2 * c].wait()
        gcp[2 * c + 1].wait()
        wx1 = wx_v[pl.ds(c * 16, 16)]
        wy1 = wy_v[pl.ds(c * 16, 16)]
        wx0 = 1.0 - wx1
        wy0 = 1.0 - wy1
        w = (wy0 * wx0, wy0 * wx1, wy1 * wx0, wy1 * wx1)
        fs = jnp.zeros((16,), jnp.float32)
        sk = jnp.zeros((16,), jnp.float32)
        for k in range(4):
            sl = pl.ds(k * 16, 16)
            fs = fs + w[k] * jnp.exp((sv_v[sl] - 1.0) * _INV_T)
            sk = sk + w[k] * mv_v[sl]
        s = sk * sc_v[pl.ds(p0, 16)]
        out_v[pl.ds(c * 32, 16)] = (1.0 - fs) * s
        out_v[pl.ds(c * 32 + 16, 16)] = s

    @pl.when(is_reproj)
    def _():
        pltpu.make_async_copy(dist_r.at[pl.ds(b_r * _N + h_r * _HN, _HN)],
                              dist_v, sem_e).wait()
        pltpu.make_async_copy(ids0_r, ids0_v, sem_f).wait()
        pltpu.make_async_copy(ids1_r, ids1_v, sem_g).wait()
        rs = jnp.zeros((16,), jnp.float32)
        rc = jnp.zeros((16,), jnp.float32)
        bvec = jnp.full((16,), b_r, jnp.int32)
        row0 = h_r * _HN
        for c in range(-(-_D // 16)):
            lidx = c * 16 + _lane_iota()
            gidx = jnp.minimum(lidx, _D - 1)
            i0 = plsc.load_gather(ids0_v, [bvec, gidx])
            i1 = plsc.load_gather(ids1_v, [bvec, gidx])
            i0l = i0 - row0
            mine = (i0l >= 0) & (i0l < _HN)
            d = plsc.load_gather(dist_v, [jnp.clip(i0l, 0, _HN - 1), i1])
            s0g = plsc.load_gather(sa_v, [i0])
            s1g = plsc.load_gather(sb_v, [i1])
            ok = (s0g > _TH) & (s1g > _TH) & mine & (lidx < _D)
            vf = jnp.where(ok, 1.0, 0.0)
            rs = rs + d * vf
            rc = rc + vf
        out_v[pl.ds(64, 16)] = rs
        out_v[pl.ds(80, 16)] = rc

    pltpu.sync_copy(out_v, part_r.at[wid])


def _fin_body(part_ref, pred0_ref, disp0_ref, pred1_ref, disp1_ref, o_ref):
    p = part_ref[...]
    q = jnp.sum(p.reshape(2, 16, 128), axis=1)

    total = jnp.float32(0.0)
    for d in range(2):
        for g in range(2):
            num_g = jnp.sum(q[g, 32 * d:32 * d + 16])
            den_g = jnp.sum(q[g, 32 * d + 16:32 * d + 32])
            total = total + num_g * jnp.float32(_N) / den_g
    loss_scoremap = total / jnp.float32(_B * 2 * _N)

    rsum = jnp.sum(q[0, 64:80]) + jnp.sum(q[1, 64:80])
    rcnt = jnp.sum(q[0, 80:96]) + jnp.sum(q[1, 80:96])
    loss_reproj = rsum / jnp.maximum(rcnt, 1.0)

    def pk(pred, disp):
        vf = jnp.where(pred > _TH, 1.0, 0.0)
        return jnp.sum(disp * vf) / jnp.maximum(jnp.sum(vf), 1.0)

    loss_peaky = (pk(pred0_ref[...], disp0_ref[...]) +
                  pk(pred1_ref[...], disp1_ref[...])) / 2.0

    o_ref[0] = (_PEAKY_W * loss_peaky + _REPROJ_W * loss_reproj +
                _SCOREMAP_W * loss_scoremap)


@jax.jit
def _detector_loss(scores_map0, scores_map1, scores_pred0, scores_pred1,
                   dispersity0, dispersity1, dist_l1, ids0_d, ids1_d,
                   scores0, scores1, kpts01, kpts10, sim01, sim10):
    sim01_f = sim01.reshape(-1)
    sim10_f = sim10.reshape(-1)
    smap0_f = scores_map0.reshape(-1)
    smap1_f = scores_map1.reshape(-1)
    dist2 = dist_l1.reshape(_B * _N, _N)
    kp = jnp.stack([kpts01, kpts10]).reshape(4, 2 * _N)
    ids0 = ids0_d.astype(jnp.int32)
    ids1 = ids1_d.astype(jnp.int32)

    mesh = plsc.VectorSubcoreMesh(core_axis_name="c", subcore_axis_name="s")
    sc_fn = pl.kernel(
        _sc_body,
        out_type=jax.ShapeDtypeStruct((_NUM_TILES, 128), jnp.float32),
        mesh=mesh,
        compiler_params=pltpu.CompilerParams(needs_layout_passes=False),
        scratch_types=[
            pltpu.VMEM((2 * _N,), jnp.float32),
            pltpu.VMEM((2 * _N,), jnp.float32),
            pltpu.VMEM((_N,), jnp.float32),
            pltpu.VMEM((_N,), jnp.float32),
            pltpu.VMEM((2 * 16,), jnp.float32),
            pltpu.VMEM((2 * 16,), jnp.float32),
            pltpu.VMEM((128,), jnp.int32),
            pltpu.VMEM((128,), jnp.int32),
            pltpu.VMEM((64,), jnp.float32),
            pltpu.VMEM((64,), jnp.float32),
            pltpu.VMEM((64,), jnp.float32),
            pltpu.VMEM((64,), jnp.float32),
            pltpu.VMEM((_B, _D), jnp.int32),
            pltpu.VMEM((_B, _D), jnp.int32),
            pltpu.VMEM((_HN, _N), jnp.float32),
            pltpu.VMEM((128,), jnp.float32),
            pltpu.SemaphoreType.DMA,
            pltpu.SemaphoreType.DMA,
            pltpu.SemaphoreType.DMA,
            pltpu.SemaphoreType.DMA,
            pltpu.SemaphoreType.DMA,
            pltpu.SemaphoreType.DMA,
            pltpu.SemaphoreType.DMA,
        ],
    )
    partials = sc_fn(sim01_f, sim10_f, smap0_f, smap1_f, kp,
                     scores0, scores1, ids0, ids1, dist2)

    loss = pl.pallas_call(
        _fin_body,
        out_shape=jax.ShapeDtypeStruct((1,), jnp.float32),
        out_specs=pl.BlockSpec(memory_space=pltpu.SMEM),
    )(partials, scores_pred0, dispersity0, scores_pred1, dispersity1)
    return loss[0]


def kernel(scores_map0, scores_map1, scores_pred0, scores_pred1, dispersity0,
           dispersity1, dist_l1, ids0_d, ids1_d, scores0, scores1, kpts01,
           kpts10, sim01, sim10):
    assert sim01.shape == (_B, _N, _H, _W)
    assert ids0_d.shape == (_B, _D)
    assert scores_pred0.shape == (_B, _M)
    return _detector_loss(scores_map0, scores_map1, scores_pred0,
                          scores_pred1, dispersity0, dispersity1, dist_l1,
                          ids0_d, ids1_d, scores0, scores1, kpts01, kpts10,
                          sim01, sim10)


# speedup vs baseline: 1.0005x; 1.0005x over previous
import functools

import jax
import jax.numpy as jnp
from jax import lax
from jax.experimental import pallas as pl
from jax.experimental.pallas import tpu as pltpu
from jax.experimental.pallas import tpu_sc as plsc

_TH = 0.1
_INV_T = 10.0
_PEAKY_W = 0.5
_REPROJ_W = 1.0
_SCOREMAP_W = 0.5

_B = 2
_N = 256
_H = 128
_W = 128
_D = 200
_M = 400

_NUM_TILES = 32
_HN = _N // 2
_PTS_PER_TILE = (_B * 2 * _N) // _NUM_TILES
_PK_PAD = 1024


def _lane_iota():
    return lax.iota(jnp.int32, 16)


def _sc_body(sim01_r, sim10_r, smap0_r, smap1_r, kp_r,
             s0_r, s1_r, ids0_r, ids1_r, dist_r,
             part_r,
             kpa_v, kpb_v, sa_v, sb_v, wx_v, wy_v,
             sidx_v, midx_v, sva_v, svb_v, mva_v, mvb_v,
             ids0_v, ids1_v, dist_v,
             out_v, sem_a, sem_b, sem_c, sem_d, sem_e, sem_f, sem_g):
    wid = lax.axis_index("c") * 16 + lax.axis_index("s")
    b = wid // 16
    p0 = (wid % 16) * 16
    is_reproj = (wid % 16) < 2
    b_r = b
    h_r = wid % 16

    zf16 = jnp.zeros((16,), jnp.float32)
    for r in range(8):
        out_v[pl.ds(r * 16, 16)] = zf16

    @pl.when(is_reproj)
    def _():
        pltpu.async_copy(dist_r.at[pl.ds(b_r * _N + h_r * _HN, _HN)], dist_v,
                         sem_e)
        pltpu.async_copy(ids0_r, ids0_v, sem_f)
        pltpu.async_copy(ids1_r, ids1_v, sem_g)

    kcp = (pltpu.async_copy(kp_r.at[b], kpa_v, sem_a),
           pltpu.async_copy(kp_r.at[2 + b], kpb_v, sem_b))
    scp = (pltpu.async_copy(s0_r.at[b], sa_v, sem_c),
           pltpu.async_copy(s1_r.at[b], sb_v, sem_d))
    nvec = p0 + _lane_iota()
    nv2 = nvec * 2
    gcp = []
    for c, kp_v in ((0, kpa_v), (1, kpb_v)):
        kcp[c].wait()
        scp[c].wait()
        kx = plsc.load_gather(kp_v, [nv2])
        ky = plsc.load_gather(kp_v, [nv2 + 1])
        x = (kx + 1.0) * (0.5 * (_W - 1))
        y = (ky + 1.0) * (0.5 * (_H - 1))
        x0 = x.astype(jnp.int32)
        y0 = y.astype(jnp.int32)
        wx_v[pl.ds(c * 16, 16)] = x - x0.astype(jnp.float32)
        wy_v[pl.ds(c * 16, 16)] = y - y0.astype(jnp.float32)
        x0c = jnp.clip(x0, 0, _W - 1)
        x1c = jnp.clip(x0 + 1, 0, _W - 1)
        y0c = jnp.clip(y0, 0, _H - 1)
        y1c = jnp.clip(y0 + 1, 0, _H - 1)
        base = (b * _N + nvec) * (_H * _W)
        row0 = base + y0c * _W
        row1 = base + y1c * _W
        sidx_v[pl.ds(c * 64 + 0, 16)] = row0 + x0c
        sidx_v[pl.ds(c * 64 + 16, 16)] = row0 + x1c
        sidx_v[pl.ds(c * 64 + 32, 16)] = row1 + x0c
        sidx_v[pl.ds(c * 64 + 48, 16)] = row1 + x1c
        mrow0 = b * (_H * _W) + y0c * _W
        mrow1 = b * (_H * _W) + y1c * _W
        midx_v[pl.ds(c * 64 + 0, 16)] = mrow0 + x0c
        midx_v[pl.ds(c * 64 + 16, 16)] = mrow0 + x1c
        midx_v[pl.ds(c * 64 + 32, 16)] = mrow1 + x0c
        midx_v[pl.ds(c * 64 + 48, 16)] = mrow1 + x1c
        if c == 0:
            gcp.append(pltpu.async_copy(sim01_r.at[sidx_v.at[pl.ds(0, 64)]],
                                        sva_v, sem_a))
            gcp.append(pltpu.async_copy(smap1_r.at[midx_v.at[pl.ds(0, 64)]],
                                        mva_v, sem_c))
        else:
            gcp.append(pltpu.async_copy(sim10_r.at[sidx_v.at[pl.ds(64, 64)]],
                                        svb_v, sem_b))
            gcp.append(pltpu.async_copy(smap0_r.at[midx_v.at[pl.ds(64, 64)]],
                                        mvb_v, sem_d))
    for c, sv_v, mv_v, sc_v in ((0, sva_v, mva_v, sa_v),
                                (1, svb_v, mvb_v, sb_v)):
        gcp[2 * c].wait()
        gcp[2 * c + 1].wait()
        wx1 = wx_v[pl.ds(c * 16, 16)]
        wy1 = wy_v[pl.ds(c * 16, 16)]
        wx0 = 1.0 - wx1
        wy0 = 1.0 - wy1
        w = (wy0 * wx0, wy0 * wx1, wy1 * wx0, wy1 * wx1)
        fs = jnp.zeros((16,), jnp.float32)
        sk = jnp.zeros((16,), jnp.float32)
        for k in range(4):
            sl = pl.ds(k * 16, 16)
            fs = fs + w[k] * jnp.exp((sv_v[sl] - 1.0) * _INV_T)
            sk = sk + w[k] * mv_v[sl]
        s = sk * sc_v[pl.ds(p0, 16)]
        out_v[pl.ds(c * 32, 16)] = (1.0 - fs) * s
        out_v[pl.ds(c * 32 + 16, 16)] = s

    @pl.when(is_reproj)
    def _():
        pltpu.make_async_copy(dist_r.at[pl.ds(b_r * _N + h_r * _HN, _HN)],
                              dist_v, sem_e).wait()
        pltpu.make_async_copy(ids0_r, ids0_v, sem_f).wait()
        pltpu.make_async_copy(ids1_r, ids1_v, sem_g).wait()
        rs = jnp.zeros((16,), jnp.float32)
        rc = jnp.zeros((16,), jnp.float32)
        bvec = jnp.full((16,), b_r, jnp.int32)
        row0 = h_r * _HN
        for c in range(-(-_D // 16)):
            lidx = c * 16 + _lane_iota()
            gidx = jnp.minimum(lidx, _D - 1)
            i0 = plsc.load_gather(ids0_v, [bvec, gidx])
            i1 = plsc.load_gather(ids1_v, [bvec, gidx])
            i0l = i0 - row0
            mine = (i0l >= 0) & (i0l < _HN)
            d = plsc.load_gather(dist_v, [jnp.clip(i0l, 0, _HN - 1), i1])
            s0g = plsc.load_gather(sa_v, [i0])
            s1g = plsc.load_gather(sb_v, [i1])
            ok = (s0g > _TH) & (s1g > _TH) & mine & (lidx < _D)
            vf = jnp.where(ok, 1.0, 0.0)
            rs = rs + d * vf
            rc = rc + vf
        out_v[pl.ds(64, 16)] = rs
        out_v[pl.ds(80, 16)] = rc

    pltpu.sync_copy(out_v, part_r.at[wid])


def _fin_body(part_ref, pred0_ref, disp0_ref, pred1_ref, disp1_ref, o_ref):
    p = part_ref[...]
    q = jnp.sum(p.reshape(2, 16, 128), axis=1)

    total = jnp.float32(0.0)
    for d in range(2):
        for g in range(2):
            num_g = jnp.sum(q[g, 32 * d:32 * d + 16])
            den_g = jnp.sum(q[g, 32 * d + 16:32 * d + 32])
            total = total + num_g * jnp.float32(_N) / den_g
    loss_scoremap = total / jnp.float32(_B * 2 * _N)

    rsum = jnp.sum(q[0, 64:80]) + jnp.sum(q[1, 64:80])
    rcnt = jnp.sum(q[0, 80:96]) + jnp.sum(q[1, 80:96])
    loss_reproj = rsum / jnp.maximum(rcnt, 1.0)

    def pk(pred, disp):
        vf = jnp.where(pred > _TH, 1.0, 0.0)
        return jnp.sum(disp * vf) / jnp.maximum(jnp.sum(vf), 1.0)

    loss_peaky = (pk(pred0_ref[...], disp0_ref[...]) +
                  pk(pred1_ref[...], disp1_ref[...])) / 2.0

    o_ref[0] = (_PEAKY_W * loss_peaky + _REPROJ_W * loss_reproj +
                _SCOREMAP_W * loss_scoremap)


@jax.jit
def _detector_loss(scores_map0, scores_map1, scores_pred0, scores_pred1,
                   dispersity0, dispersity1, dist_l1, ids0_d, ids1_d,
                   scores0, scores1, kpts01, kpts10, sim01, sim10):
    sim01_f = sim01.reshape(-1)
    sim10_f = sim10.reshape(-1)
    smap0_f = scores_map0.reshape(-1)
    smap1_f = scores_map1.reshape(-1)
    dist2 = dist_l1.reshape(_B * _N, _N)
    kp = jnp.stack([kpts01, kpts10]).reshape(4, 2 * _N)
    ids0 = ids0_d.astype(jnp.int32)
    ids1 = ids1_d.astype(jnp.int32)

    mesh = plsc.VectorSubcoreMesh(core_axis_name="c", subcore_axis_name="s")
    sc_fn = pl.kernel(
        _sc_body,
        out_type=jax.ShapeDtypeStruct((_NUM_TILES, 128), jnp.float32),
        mesh=mesh,
        compiler_params=pltpu.CompilerParams(needs_layout_passes=False, skip_device_barrier=True),
        scratch_types=[
            pltpu.VMEM((2 * _N,), jnp.float32),
            pltpu.VMEM((2 * _N,), jnp.float32),
            pltpu.VMEM((_N,), jnp.float32),
            pltpu.VMEM((_N,), jnp.float32),
            pltpu.VMEM((2 * 16,), jnp.float32),
            pltpu.VMEM((2 * 16,), jnp.float32),
            pltpu.VMEM((128,), jnp.int32),
            pltpu.VMEM((128,), jnp.int32),
            pltpu.VMEM((64,), jnp.float32),
            pltpu.VMEM((64,), jnp.float32),
            pltpu.VMEM((64,), jnp.float32),
            pltpu.VMEM((64,), jnp.float32),
            pltpu.VMEM((_B, _D), jnp.int32),
            pltpu.VMEM((_B, _D), jnp.int32),
            pltpu.VMEM((_HN, _N), jnp.float32),
            pltpu.VMEM((128,), jnp.float32),
            pltpu.SemaphoreType.DMA,
            pltpu.SemaphoreType.DMA,
            pltpu.SemaphoreType.DMA,
            pltpu.SemaphoreType.DMA,
            pltpu.SemaphoreType.DMA,
            pltpu.SemaphoreType.DMA,
            pltpu.SemaphoreType.DMA,
        ],
    )
    partials = sc_fn(sim01_f, sim10_f, smap0_f, smap1_f, kp,
                     scores0, scores1, ids0, ids1, dist2)

    loss = pl.pallas_call(
        _fin_body,
        out_shape=jax.ShapeDtypeStruct((1,), jnp.float32),
        out_specs=pl.BlockSpec(memory_space=pltpu.SMEM),
    )(partials, scores_pred0, dispersity0, scores_pred1, dispersity1)
    return loss[0]


def kernel(scores_map0, scores_map1, scores_pred0, scores_pred1, dispersity0,
           dispersity1, dist_l1, ids0_d, ids1_d, scores0, scores1, kpts01,
           kpts10, sim01, sim10):
    assert sim01.shape == (_B, _N, _H, _W)
    assert ids0_d.shape == (_B, _D)
    assert scores_pred0.shape == (_B, _M)
    return _detector_loss(scores_map0, scores_map1, scores_pred0,
                          scores_pred1, dispersity0, dispersity1, dist_l1,
                          ids0_d, ids1_d, scores0, scores1, kpts01, kpts10,
                          sim01, sim10)


# speedup vs baseline: 1.0039x; 1.0034x over previous
import functools

import jax
import jax.numpy as jnp
from jax import lax
from jax.experimental import pallas as pl
from jax.experimental.pallas import tpu as pltpu
from jax.experimental.pallas import tpu_sc as plsc

_TH = 0.1
_INV_T = 10.0
_PEAKY_W = 0.5
_REPROJ_W = 1.0
_SCOREMAP_W = 0.5

_B = 2
_N = 256
_H = 128
_W = 128
_D = 200
_M = 400

_NUM_TILES = 32
_HN = _N // 2
_PTS_PER_TILE = (_B * 2 * _N) // _NUM_TILES
_PK_PAD = 1024


def _lane_iota():
    return lax.iota(jnp.int32, 16)


def _sc_body(sim01_r, sim10_r, smap0_r, smap1_r, kp_r,
             s0_r, s1_r, ids0_r, ids1_r, dist_r,
             part_r,
             kpa_v, kpb_v, sa_v, sb_v, wx_v, wy_v,
             sidx_v, midx_v, sva_v, svb_v, mva_v, mvb_v,
             ids0_v, ids1_v, dist_v,
             out_v, sem_a, sem_b, sem_c, sem_d, sem_e, sem_f, sem_g):
    wid = lax.axis_index("c") * 16 + lax.axis_index("s")
    b = wid // 16
    p0 = (wid % 16) * 16
    is_reproj = (wid % 16) < 2
    b_r = b
    h_r = wid % 16

    zf16 = jnp.zeros((16,), jnp.float32)
    for r in range(8):
        out_v[pl.ds(r * 16, 16)] = zf16

    @pl.when(is_reproj)
    def _():
        pltpu.async_copy(dist_r.at[pl.ds(b_r * _N + h_r * _HN, _HN)], dist_v,
                         sem_e)
        pltpu.async_copy(ids0_r, ids0_v, sem_f)
        pltpu.async_copy(ids1_r, ids1_v, sem_g)

    kcp = (pltpu.async_copy(kp_r.at[b], kpa_v, sem_a),
           pltpu.async_copy(kp_r.at[2 + b], kpb_v, sem_b))
    scp = (pltpu.async_copy(s0_r.at[b], sa_v, sem_c),
           pltpu.async_copy(s1_r.at[b], sb_v, sem_d))
    nvec = p0 + _lane_iota()
    nv2 = nvec * 2
    gcp = []
    for c, kp_v in ((0, kpa_v), (1, kpb_v)):
        kcp[c].wait()
        scp[c].wait()
        kx = plsc.load_gather(kp_v, [nv2])
        ky = plsc.load_gather(kp_v, [nv2 + 1])
        x = (kx + 1.0) * (0.5 * (_W - 1))
        y = (ky + 1.0) * (0.5 * (_H - 1))
        x0 = x.astype(jnp.int32)
        y0 = y.astype(jnp.int32)
        wx_v[pl.ds(c * 16, 16)] = x - x0.astype(jnp.float32)
        wy_v[pl.ds(c * 16, 16)] = y - y0.astype(jnp.float32)
        x0c = jnp.clip(x0, 0, _W - 1)
        x1c = jnp.clip(x0 + 1, 0, _W - 1)
        y0c = jnp.clip(y0, 0, _H - 1)
        y1c = jnp.clip(y0 + 1, 0, _H - 1)
        base = (b * _N + nvec) * (_H * _W)
        row0 = base + y0c * _W
        row1 = base + y1c * _W
        sidx_v[pl.ds(c * 64 + 0, 16)] = row0 + x0c
        sidx_v[pl.ds(c * 64 + 16, 16)] = row0 + x1c
        sidx_v[pl.ds(c * 64 + 32, 16)] = row1 + x0c
        sidx_v[pl.ds(c * 64 + 48, 16)] = row1 + x1c
        mrow0 = b * (_H * _W) + y0c * _W
        mrow1 = b * (_H * _W) + y1c * _W
        midx_v[pl.ds(c * 64 + 0, 16)] = mrow0 + x0c
        midx_v[pl.ds(c * 64 + 16, 16)] = mrow0 + x1c
        midx_v[pl.ds(c * 64 + 32, 16)] = mrow1 + x0c
        midx_v[pl.ds(c * 64 + 48, 16)] = mrow1 + x1c
        if c == 0:
            gcp.append(pltpu.async_copy(sim01_r.at[sidx_v.at[pl.ds(0, 64)]],
                                        sva_v, sem_a))
            gcp.append(pltpu.async_copy(smap1_r.at[midx_v.at[pl.ds(0, 64)]],
                                        mva_v, sem_c))
        else:
            gcp.append(pltpu.async_copy(sim10_r.at[sidx_v.at[pl.ds(64, 64)]],
                                        svb_v, sem_b))
            gcp.append(pltpu.async_copy(smap0_r.at[midx_v.at[pl.ds(64, 64)]],
                                        mvb_v, sem_d))
    for c, sv_v, mv_v, sc_v in ((0, sva_v, mva_v, sa_v),
                                (1, svb_v, mvb_v, sb_v)):
        gcp[2 * c].wait()
        gcp[2 * c + 1].wait()
        wx1 = wx_v[pl.ds(c * 16, 16)]
        wy1 = wy_v[pl.ds(c * 16, 16)]
        wx0 = 1.0 - wx1
        wy0 = 1.0 - wy1
        w = (wy0 * wx0, wy0 * wx1, wy1 * wx0, wy1 * wx1)
        fs = jnp.zeros((16,), jnp.float32)
        sk = jnp.zeros((16,), jnp.float32)
        for k in range(4):
            sl = pl.ds(k * 16, 16)
            fs = fs + w[k] * jnp.exp((sv_v[sl] - 1.0) * _INV_T)
            sk = sk + w[k] * mv_v[sl]
        s = sk * sc_v[pl.ds(p0, 16)]
        out_v[pl.ds(c * 32, 16)] = (1.0 - fs) * s
        out_v[pl.ds(c * 32 + 16, 16)] = s

    @pl.when(is_reproj)
    def _():
        pltpu.make_async_copy(dist_r.at[pl.ds(b_r * _N + h_r * _HN, _HN)],
                              dist_v, sem_e).wait()
        pltpu.make_async_copy(ids0_r, ids0_v, sem_f).wait()
        pltpu.make_async_copy(ids1_r, ids1_v, sem_g).wait()
        bvec = jnp.full((16,), b_r, jnp.int32)
        row0 = h_r * _HN

        def rchunk(c, carry):
            rs, rc = carry
            lidx = c * 16 + _lane_iota()
            gidx = jnp.minimum(lidx, _D - 1)
            i0 = plsc.load_gather(ids0_v, [bvec, gidx])
            i1 = plsc.load_gather(ids1_v, [bvec, gidx])
            i0l = i0 - row0
            mine = (i0l >= 0) & (i0l < _HN)
            d = plsc.load_gather(dist_v, [jnp.clip(i0l, 0, _HN - 1), i1])
            s0g = plsc.load_gather(sa_v, [i0])
            s1g = plsc.load_gather(sb_v, [i1])
            ok = (s0g > _TH) & (s1g > _TH) & mine & (lidx < _D)
            vf = jnp.where(ok, 1.0, 0.0)
            return rs + d * vf, rc + vf

        rs, rc = lax.fori_loop(0, -(-_D // 16),
                               rchunk, (jnp.zeros((16,), jnp.float32),
                                        jnp.zeros((16,), jnp.float32)))
        out_v[pl.ds(64, 16)] = rs
        out_v[pl.ds(80, 16)] = rc

    pltpu.sync_copy(out_v, part_r.at[wid])


def _fin_body(part_ref, pred0_ref, disp0_ref, pred1_ref, disp1_ref, o_ref):
    p = part_ref[...]
    q = jnp.sum(p.reshape(2, 16, 128), axis=1)

    total = jnp.float32(0.0)
    for d in range(2):
        for g in range(2):
            num_g = jnp.sum(q[g, 32 * d:32 * d + 16])
            den_g = jnp.sum(q[g, 32 * d + 16:32 * d + 32])
            total = total + num_g * jnp.float32(_N) / den_g
    loss_scoremap = total / jnp.float32(_B * 2 * _N)

    rsum = jnp.sum(q[0, 64:80]) + jnp.sum(q[1, 64:80])
    rcnt = jnp.sum(q[0, 80:96]) + jnp.sum(q[1, 80:96])
    loss_reproj = rsum / jnp.maximum(rcnt, 1.0)

    def pk(pred, disp):
        vf = jnp.where(pred > _TH, 1.0, 0.0)
        return jnp.sum(disp * vf) / jnp.maximum(jnp.sum(vf), 1.0)

    loss_peaky = (pk(pred0_ref[...], disp0_ref[...]) +
                  pk(pred1_ref[...], disp1_ref[...])) / 2.0

    o_ref[0] = (_PEAKY_W * loss_peaky + _REPROJ_W * loss_reproj +
                _SCOREMAP_W * loss_scoremap)


@jax.jit
def _detector_loss(scores_map0, scores_map1, scores_pred0, scores_pred1,
                   dispersity0, dispersity1, dist_l1, ids0_d, ids1_d,
                   scores0, scores1, kpts01, kpts10, sim01, sim10):
    sim01_f = sim01.reshape(-1)
    sim10_f = sim10.reshape(-1)
    smap0_f = scores_map0.reshape(-1)
    smap1_f = scores_map1.reshape(-1)
    dist2 = dist_l1.reshape(_B * _N, _N)
    kp = jnp.stack([kpts01, kpts10]).reshape(4, 2 * _N)
    ids0 = ids0_d.astype(jnp.int32)
    ids1 = ids1_d.astype(jnp.int32)

    mesh = plsc.VectorSubcoreMesh(core_axis_name="c", subcore_axis_name="s")
    sc_fn = pl.kernel(
        _sc_body,
        out_type=jax.ShapeDtypeStruct((_NUM_TILES, 128), jnp.float32),
        mesh=mesh,
        compiler_params=pltpu.CompilerParams(needs_layout_passes=False),
        scratch_types=[
            pltpu.VMEM((2 * _N,), jnp.float32),
            pltpu.VMEM((2 * _N,), jnp.float32),
            pltpu.VMEM((_N,), jnp.float32),
            pltpu.VMEM((_N,), jnp.float32),
            pltpu.VMEM((2 * 16,), jnp.float32),
            pltpu.VMEM((2 * 16,), jnp.float32),
            pltpu.VMEM((128,), jnp.int32),
            pltpu.VMEM((128,), jnp.int32),
            pltpu.VMEM((64,), jnp.float32),
            pltpu.VMEM((64,), jnp.float32),
            pltpu.VMEM((64,), jnp.float32),
            pltpu.VMEM((64,), jnp.float32),
            pltpu.VMEM((_B, _D), jnp.int32),
            pltpu.VMEM((_B, _D), jnp.int32),
            pltpu.VMEM((_HN, _N), jnp.float32),
            pltpu.VMEM((128,), jnp.float32),
            pltpu.SemaphoreType.DMA,
            pltpu.SemaphoreType.DMA,
            pltpu.SemaphoreType.DMA,
            pltpu.SemaphoreType.DMA,
            pltpu.SemaphoreType.DMA,
            pltpu.SemaphoreType.DMA,
            pltpu.SemaphoreType.DMA,
        ],
    )
    partials = sc_fn(sim01_f, sim10_f, smap0_f, smap1_f, kp,
                     scores0, scores1, ids0, ids1, dist2)

    loss = pl.pallas_call(
        _fin_body,
        out_shape=jax.ShapeDtypeStruct((1,), jnp.float32),
        out_specs=pl.BlockSpec(memory_space=pltpu.SMEM),
    )(partials, scores_pred0, dispersity0, scores_pred1, dispersity1)
    return loss[0]


def kernel(scores_map0, scores_map1, scores_pred0, scores_pred1, dispersity0,
           dispersity1, dist_l1, ids0_d, ids1_d, scores0, scores1, kpts01,
           kpts10, sim01, sim10):
    assert sim01.shape == (_B, _N, _H, _W)
    assert ids0_d.shape == (_B, _D)
    assert scores_pred0.shape == (_B, _M)
    return _detector_loss(scores_map0, scores_map1, scores_pred0,
                          scores_pred1, dispersity0, dispersity1, dist_l1,
                          ids0_d, ids1_d, scores0, scores1, kpts01, kpts10,
                          sim01, sim10)
